# baseline (speedup 1.0000x reference)
import dataclasses
import functools

import jax
import jax.numpy as jnp
from jax import lax
from jax.experimental import pallas as pl
from jax.experimental.pallas import tpu as pltpu
from jax.experimental.pallas import tpu_sc as plsc

K = 100000
D = 128
NC = 2
NS = 16
NW = NC * NS
RPW = K // NW
NCHUNK = 25
CH = RPW // NCHUNK
U = 5
NV = D // 16


def _sc_l1_argmin(keys, query_flat):
    mesh = plsc.VectorSubcoreMesh(core_axis_name="c", subcore_axis_name="s")
    cp = pltpu.CompilerParams()
    if "needs_layout_passes" in pltpu.CompilerParams.__dataclass_fields__:
        cp = dataclasses.replace(cp, needs_layout_passes=False)

    @functools.partial(
        pl.kernel,
        mesh=mesh,
        compiler_params=cp,
        out_type=[
            jax.ShapeDtypeStruct((NW, 16), jnp.float32),
            jax.ShapeDtypeStruct((NW, 16), jnp.int32),
        ],
        scratch_types=[
            pltpu.VMEM((CH * D,), jnp.float32),
            pltpu.VMEM((CH * D,), jnp.float32),
            pltpu.VMEM((D,), jnp.float32),
            pltpu.VMEM((16,), jnp.float32),
            pltpu.VMEM((16,), jnp.int32),
            pltpu.SemaphoreType.DMA,
            pltpu.SemaphoreType.DMA,
        ],
    )
    def k(keys_hbm, q_hbm, out_v_hbm, out_i_hbm,
          buf0, buf1, q_v, res_v, resi_v, sem0, sem1):
        wid = lax.axis_index("s") * NC + lax.axis_index("c")
        base = wid * RPW
        pltpu.sync_copy(q_hbm, q_v)
        qs = [q_v[pl.ds(16 * j, 16)] for j in range(NV)]
        bufs = (buf0, buf1)
        sems = (sem0, sem1)

        def start(g, slot):
            return pltpu.async_copy(
                keys_hbm.at[pl.ds((base + g * CH) * D, CH * D)],
                bufs[slot], sems[slot])

        handles = {0: start(0, 0), 1: start(1, 1)}
        bv = jnp.float32(jnp.inf)
        bi = jnp.int32(0)

        for g in range(NCHUNK):
            slot = g % 2
            handles.pop(g).wait()
            buf = bufs[slot]
            gbase = base + g * CH

            def body(i, carry, buf=buf, gbase=gbase):
                bv, bi = carry
                for u in range(U):
                    r = i * U + u
                    d = [jnp.abs(buf[pl.ds(r * D + 16 * j, 16)] - qs[j])
                         for j in range(NV)]
                    s1 = [d[0] + d[1], d[2] + d[3], d[4] + d[5], d[6] + d[7]]
                    acc = (s1[0] + s1[1]) + (s1[2] + s1[3])
                    s = jnp.sum(acc)
                    pred = s < bv
                    bv = jnp.where(pred, s, bv)
                    bi = jnp.where(pred, gbase + r, bi)
                return bv, bi

            bv, bi = lax.fori_loop(0, CH // U, body, (bv, bi))
            if g + 2 < NCHUNK:
                handles[g + 2] = start(g + 2, slot)

        res_v[...] = jnp.full((16,), bv, jnp.float32)
        resi_v[...] = jnp.full((16,), bi, jnp.int32)
        pltpu.sync_copy(res_v, out_v_hbm.at[wid])
        pltpu.sync_copy(resi_v, out_i_hbm.at[wid])

    return k(keys, query_flat)


def kernel(keys, query):
    vals, idxs = _sc_l1_argmin(keys.reshape((K * D,)), query.reshape((D,)))
    v = vals[:, 0]
    i = idxs[:, 0]
    return i[jnp.argmin(v)]


# speedup vs baseline: 1.2809x; 1.2809x over previous
import dataclasses
import functools

import jax
import jax.numpy as jnp
from jax import lax
from jax.experimental import pallas as pl
from jax.experimental.pallas import tpu as pltpu
from jax.experimental.pallas import tpu_sc as plsc

K = 100000
D = 128
NC = 2
NS = 16
NW = NC * NS
RPW = K // NW
NBUF = 5
NROUND = 5
CH = RPW // (NBUF * NROUND)
U = 5
NV = D // 16


def _sc_l1_argmin(keys, query_flat):
    mesh = plsc.VectorSubcoreMesh(core_axis_name="c", subcore_axis_name="s")
    cp = pltpu.CompilerParams()
    if "needs_layout_passes" in pltpu.CompilerParams.__dataclass_fields__:
        cp = dataclasses.replace(cp, needs_layout_passes=False)

    @functools.partial(
        pl.kernel,
        mesh=mesh,
        compiler_params=cp,
        out_type=[
            jax.ShapeDtypeStruct((NW, 16), jnp.float32),
            jax.ShapeDtypeStruct((NW, 16), jnp.int32),
        ],
        scratch_types=[pltpu.VMEM((CH * D,), jnp.float32)] * NBUF + [
            pltpu.VMEM((D,), jnp.float32),
            pltpu.VMEM((16,), jnp.float32),
            pltpu.VMEM((16,), jnp.int32),
        ] + [pltpu.SemaphoreType.DMA] * NBUF,
    )
    def k(keys_hbm, q_hbm, out_v_hbm, out_i_hbm, *rest):
        bufs = rest[:NBUF]
        q_v, res_v, resi_v = rest[NBUF:NBUF + 3]
        sems = rest[NBUF + 3:]
        wid = lax.axis_index("s") * NC + lax.axis_index("c")
        base = wid * RPW
        pltpu.sync_copy(q_hbm, q_v)
        qs = [q_v[pl.ds(16 * j, 16)] for j in range(NV)]

        def start(g, b):
            pltpu.async_copy(
                keys_hbm.at[pl.ds((base + g * CH) * D, CH * D)],
                bufs[b], sems[b])

        def wait(b):
            pltpu.make_async_copy(
                keys_hbm.at[pl.ds(0, CH * D)], bufs[b], sems[b]).wait()

        for b in range(NBUF):
            start(b, b)

        def chunk_body(i, carry, b):
            bv, bi = carry
            g = i * NBUF + b
            wait(b)
            gbase = base + g * CH

            def body(r5, carry):
                bv, bi = carry
                for u in range(U):
                    r = r5 * U + u
                    d = [jnp.abs(bufs[b][pl.ds(r * D + 16 * j, 16)] - qs[j])
                         for j in range(NV)]
                    s1 = [d[0] + d[1], d[2] + d[3], d[4] + d[5], d[6] + d[7]]
                    acc = (s1[0] + s1[1]) + (s1[2] + s1[3])
                    s = jnp.sum(acc)
                    pred = s < bv
                    bv = jnp.where(pred, s, bv)
                    bi = jnp.where(pred, gbase + r, bi)
                return bv, bi

            bv, bi = lax.fori_loop(0, CH // U, body, (bv, bi))

            @pl.when(i < NROUND - 1)
            def _():
                start(g + NBUF, b)

            return bv, bi

        def round_body(i, carry):
            for b in range(NBUF):
                carry = chunk_body(i, carry, b)
            return carry

        bv, bi = lax.fori_loop(
            0, NROUND, round_body, (jnp.float32(jnp.inf), jnp.int32(0)))

        res_v[...] = jnp.full((16,), bv, jnp.float32)
        resi_v[...] = jnp.full((16,), bi, jnp.int32)
        pltpu.sync_copy(res_v, out_v_hbm.at[wid])
        pltpu.sync_copy(resi_v, out_i_hbm.at[wid])

    return k(keys, query_flat)


def kernel(keys, query):
    vals, idxs = _sc_l1_argmin(keys.reshape((K * D,)), query.reshape((D,)))
    v = vals[:, 0]
    i = idxs[:, 0]
    return i[jnp.argmin(v)]
